# Initial kernel scaffold; baseline (speedup 1.0000x reference)
import jax
import jax.numpy as jnp
from jax.experimental import pallas as pl


def kernel(x, edge_index, edge_weight, gamma0, beta0, W1, b1, gamma1, beta1, Wrel2, brel2, Wroot2, gamma2, beta2, Wrel3, brel3, Wroot3, gamma3, beta3):
    raise NotImplementedError("write your pallas kernel here")


# speedup vs baseline: 5.8139x; 5.8139x over previous
import functools

import jax
import jax.numpy as jnp
from jax import lax
from jax.experimental import pallas as pl
from jax.experimental.pallas import tpu as pltpu
from jax.experimental.pallas import tpu_sc as plsc

N = 10000
E = 320000
D = 128
NC = 2
NS = 16
NW = NC * NS
L = 16

K = 128
EPW = -(-E // NW)
NCHUNK = -(-EPW // K)
E_PAD = NW * NCHUNK * K
N_ACC = 10240
RPT = N_ACC // NS
RCH = RPT // K

_mesh = plsc.VectorSubcoreMesh(core_axis_name="c", subcore_axis_name="s")


def _zero_rows_buf(buf):
    zz = jnp.zeros((L,), jnp.float32)

    def zb(i, _):
        for c in range(D // L):
            buf[i, pl.ds(c * L, L)] = zz
        return 0

    lax.fori_loop(0, K, zb, 0, unroll=False)


@functools.partial(
    pl.kernel,
    out_type=jax.ShapeDtypeStruct((NC, N_ACC), jnp.float32),
    mesh=_mesh,
    scratch_types=[
        pltpu.VMEM((K,), jnp.int32),
        pltpu.VMEM((K,), jnp.float32),
        pltpu.VMEM((RPT,), jnp.float32),
        pltpu.VMEM_SHARED((N_ACC,), jnp.float32),
    ],
)
def _deg_kernel(col_hbm, w_hbm, out_hbm, col_v, w_v, zbuf, acc_sh):
    cid = lax.axis_index("c")
    sid = lax.axis_index("s")
    wid = sid * NC + cid

    zz = jnp.zeros((L,), jnp.float32)

    def zb(i, _):
        zbuf[pl.ds(i * L, L)] = zz
        return 0

    lax.fori_loop(0, RPT // L, zb, 0, unroll=False)
    pltpu.sync_copy(zbuf, acc_sh.at[pl.ds(sid * RPT, RPT)])
    plsc.subcore_barrier()

    def body(j, _):
        pltpu.sync_copy(col_hbm.at[wid, j], col_v)
        pltpu.sync_copy(w_hbm.at[wid, j], w_v)
        pltpu.sync_copy(w_v, acc_sh.at[col_v], add=True)
        return 0

    lax.fori_loop(0, NCHUNK, body, 0, unroll=False)
    plsc.subcore_barrier()
    pltpu.sync_copy(acc_sh.at[pl.ds(sid * RPT, RPT)],
                    out_hbm.at[cid, pl.ds(sid * RPT, RPT)])


def _make_agg(with_dis):
    scratch = [
        pltpu.VMEM((K,), jnp.int32),
        pltpu.VMEM((K,), jnp.int32),
        pltpu.VMEM((K,), jnp.float32),
        pltpu.VMEM((K, D), jnp.float32),
        pltpu.VMEM_SHARED((N_ACC, D), jnp.float32),
        pltpu.SemaphoreType.DMA,
    ]
    if with_dis:
        scratch.append(pltpu.VMEM((N_ACC,), jnp.float32))

    def body(*refs):
        if with_dis:
            (row_hbm, col_hbm, w_hbm, dis_hbm, src_hbm, out_hbm,
             row_v, col_v, w_v, rows_v, acc_sh, sem, dis_v) = refs
        else:
            (row_hbm, col_hbm, w_hbm, src_hbm, out_hbm,
             row_v, col_v, w_v, rows_v, acc_sh, sem) = refs

        cid = lax.axis_index("c")
        sid = lax.axis_index("s")
        wid = sid * NC + cid

        if with_dis:
            pltpu.sync_copy(dis_hbm, dis_v)

        _zero_rows_buf(rows_v)
        for i in range(RCH):
            pltpu.sync_copy(rows_v, acc_sh.at[pl.ds(sid * RPT + i * K, K)])
        plsc.subcore_barrier()

        def chunk(j, _):
            pltpu.sync_copy(row_hbm.at[wid, j], row_v)
            pltpu.sync_copy(col_hbm.at[wid, j], col_v)
            pltpu.sync_copy(w_hbm.at[wid, j], w_v)
            if with_dis:
                for g in range(K // L):
                    sl = pl.ds(g * L, L)
                    dr = plsc.load_gather(dis_v, [row_v[sl]])
                    dc = plsc.load_gather(dis_v, [col_v[sl]])
                    w_v[sl] = w_v[sl] * dr * dc
            pltpu.async_copy(src_hbm.at[row_v], rows_v, sem).wait()

            def scale(e, _):
                wv = plsc.load_gather(w_v, [jnp.full((L,), e, jnp.int32)])
                for c in range(D // L):
                    sl = pl.ds(c * L, L)
                    rows_v[e, sl] = rows_v[e, sl] * wv
                return 0

            lax.fori_loop(0, K, scale, 0, unroll=False)
            pltpu.sync_copy(rows_v, acc_sh.at[col_v], add=True)
            return 0

        lax.fori_loop(0, NCHUNK, chunk, 0, unroll=False)
        plsc.subcore_barrier()
        for i in range(RCH):
            r0 = sid * RPT + i * K
            pltpu.sync_copy(acc_sh.at[pl.ds(r0, K)],
                            out_hbm.at[cid, pl.ds(r0, K)])

    return pl.kernel(
        body,
        out_type=jax.ShapeDtypeStruct((NC, N_ACC, D), jnp.float32),
        mesh=_mesh,
        scratch_types=scratch,
        compiler_params=pltpu.CompilerParams(needs_layout_passes=False),
    )


_agg_plain = _make_agg(False)
_agg_gcn = _make_agg(True)


def _bn(h, gamma, beta):
    m = jnp.mean(h, axis=0)
    v = jnp.mean(h * h, axis=0) - m * m
    return (h - m) * lax.rsqrt(v + 1e-5) * gamma + beta


def _tc_call(fn, out_shapes, *args):
    return pl.pallas_call(
        fn,
        out_shape=[jax.ShapeDtypeStruct(s, jnp.float32) for s in out_shapes],
    )(*args)


def _bn0_mm_body(x_ref, g_ref, b_ref, w_ref, h1_ref):
    xn = _bn(x_ref[...], g_ref[...], b_ref[...])
    h1_ref[...] = jnp.dot(xn, w_ref[...], preferred_element_type=jnp.float32)


def _dis_body(degp_ref, dis_ref):
    deg = degp_ref[0] + degp_ref[1] + 1.0
    dis_ref[...] = lax.rsqrt(jnp.maximum(deg, 1e-12))


def _layer1_body(p0_ref, p1_ref, h1_ref, dis_ref, b1_ref, g1_ref, be1_ref,
                 wr_ref, br_ref, wo_ref, t_ref, r_ref):
    dis = dis_ref[...]
    o1 = (p0_ref[...] + p1_ref[...] + h1_ref[...] * (dis * dis)
          + b1_ref[...])
    h2 = _bn(jnp.maximum(o1, 0.0), g1_ref[...], be1_ref[...])
    t_ref[...] = jnp.dot(h2, wr_ref[...], preferred_element_type=jnp.float32)
    r_ref[...] = (jnp.dot(h2, wo_ref[...], preferred_element_type=jnp.float32)
                  + br_ref[...])


def _layer2_body(p0_ref, p1_ref, r_in_ref, g_ref, be_ref,
                 wr_ref, br_ref, wo_ref, t_ref, r_ref):
    o = p0_ref[...] + p1_ref[...] + r_in_ref[...]
    h = _bn(jnp.maximum(o, 0.0), g_ref[...], be_ref[...])
    t_ref[...] = jnp.dot(h, wr_ref[...], preferred_element_type=jnp.float32)
    r_ref[...] = (jnp.dot(h, wo_ref[...], preferred_element_type=jnp.float32)
                  + br_ref[...])


def _final_body(p0_ref, p1_ref, r_in_ref, g_ref, be_ref, out_ref):
    o = p0_ref[...] + p1_ref[...] + r_in_ref[...]
    out_ref[...] = _bn(jnp.maximum(o, 0.0), g_ref[...], be_ref[...])


def kernel(x, edge_index, edge_weight, gamma0, beta0, W1, b1, gamma1, beta1,
           Wrel2, brel2, Wroot2, gamma2, beta2,
           Wrel3, brel3, Wroot3, gamma3, beta3):
    pad = E_PAD - E
    row = jnp.pad(edge_index[0].astype(jnp.int32), (0, pad))
    col = jnp.pad(edge_index[1].astype(jnp.int32), (0, pad))
    w = jnp.pad(edge_weight, (0, pad))
    row3 = row.reshape(NW, NCHUNK, K)
    col3 = col.reshape(NW, NCHUNK, K)
    w3 = w.reshape(NW, NCHUNK, K)

    degp = _deg_kernel(col3, w3)
    h1, = _tc_call(_bn0_mm_body, [(N, D)], x, gamma0, beta0, W1)

    dis, = _tc_call(_dis_body, [(N_ACC,)], degp)
    dis_col = dis[:N].reshape(N, 1)

    agg1 = _agg_gcn(row3, col3, w3, dis, h1)
    t2, r2 = _tc_call(_layer1_body, [(N, D), (N, D)],
                      agg1[0, :N], agg1[1, :N], h1, dis_col,
                      b1, gamma1, beta1, Wrel2, brel2, Wroot2)

    agg2 = _agg_plain(row3, col3, w3, t2)
    t3, r3 = _tc_call(_layer2_body, [(N, D), (N, D)],
                      agg2[0, :N], agg2[1, :N], r2, gamma2, beta2,
                      Wrel3, brel3, Wroot3)

    agg3 = _agg_plain(row3, col3, w3, t3)
    out, = _tc_call(_final_body, [(N, D)],
                    agg3[0, :N], agg3[1, :N], r3, gamma3, beta3)
    return out
